# baseline (speedup 1.0000x reference)
import functools

import jax
import jax.numpy as jnp
from jax import lax
from jax.experimental import pallas as pl
from jax.experimental.pallas import tpu as pltpu
from jax.experimental.pallas import tpu_sc as plsc

NUM_GENRES = 1000000
EMB_DIM = 10

_BL = 131072


def _proj_body(w_ref, b_ref, tab_ref, out_ref):
    t = tab_ref[...]
    w = w_ref[...]
    out_ref[...] = jnp.sum(t * w, axis=0) + b_ref[0]


def _project(table, fc_w, fc_b):
    tab_t = table.T
    grid = (pl.cdiv(NUM_GENRES, _BL),)
    return pl.pallas_call(
        _proj_body,
        grid=grid,
        in_specs=[
            pl.BlockSpec((EMB_DIM, 1), lambda i: (0, 0)),
            pl.BlockSpec(memory_space=pltpu.SMEM),
            pl.BlockSpec((EMB_DIM, _BL), lambda i: (0, i)),
        ],
        out_specs=pl.BlockSpec((_BL,), lambda i: (i,)),
        out_shape=jax.ShapeDtypeStruct((NUM_GENRES,), jnp.float32),
    )(fc_w.reshape(EMB_DIM, 1), fc_b, tab_t)


def _make_gather(n_idx):
    info = plsc.get_sparse_core_info()
    nw = info.num_cores * info.num_subcores
    per_w = n_idx // nw
    chunk = 6400
    n_chunks = per_w // chunk
    mesh = plsc.VectorSubcoreMesh(core_axis_name="c", subcore_axis_name="s")

    @functools.partial(
        pl.kernel,
        mesh=mesh,
        out_type=jax.ShapeDtypeStruct((n_idx,), jnp.float32),
        scratch_types=[
            pltpu.VMEM_SHARED((NUM_GENRES,), jnp.float32),
            pltpu.VMEM((chunk,), jnp.int32),
            pltpu.VMEM((chunk,), jnp.int32),
            pltpu.VMEM((chunk,), jnp.float32),
            pltpu.VMEM((chunk,), jnp.float32),
            pltpu.SemaphoreType.DMA,
            pltpu.SemaphoreType.DMA,
            pltpu.SemaphoreType.DMA,
            pltpu.SemaphoreType.DMA,
            pltpu.SemaphoreType.DMA,
        ],
    )
    def gather_k(idx_hbm, proj_hbm, out_hbm,
                 shared, idx_v0, idx_v1, val_v0, val_v1,
                 sem_i0, sem_i1, sem_g, sem_o0, sem_o1):
        sid = lax.axis_index("s")
        wid = sid * info.num_cores + lax.axis_index("c")
        base = wid * per_w
        idx_v = (idx_v0, idx_v1)
        val_v = (val_v0, val_v1)
        sem_i = (sem_i0, sem_i1)
        sem_o = (sem_o0, sem_o1)

        idx_h = [None, None]
        out_h = [None, None]
        idx_h[0] = pltpu.async_copy(
            idx_hbm.at[pl.ds(base, chunk)], idx_v[0], sem_i[0])

        @pl.when(sid == 0)
        def _stage():
            pltpu.sync_copy(proj_hbm, shared)

        plsc.subcore_barrier()

        for i in range(n_chunks):
            cur = i & 1
            nxt = 1 - cur
            idx_h[cur].wait()
            if i + 1 < n_chunks:
                idx_h[nxt] = pltpu.async_copy(
                    idx_hbm.at[pl.ds(base + (i + 1) * chunk, chunk)],
                    idx_v[nxt], sem_i[nxt])
            if out_h[cur] is not None:
                out_h[cur].wait()
            pltpu.async_copy(shared.at[idx_v[cur]], val_v[cur], sem_g).wait()
            out_h[cur] = pltpu.async_copy(
                val_v[cur], out_hbm.at[pl.ds(base + i * chunk, chunk)],
                sem_o[cur])
        out_h[0].wait()
        out_h[1].wait()

    return gather_k


def kernel(movie_ids, embedding_table, fc_w, fc_b):
    b, l = movie_ids.shape
    n = b * l
    proj = _project(embedding_table, fc_w, fc_b)
    ids_lin = movie_ids.T.reshape(-1)
    out = _make_gather(n)(ids_lin, proj)
    return out.reshape(l, b, 1).transpose(1, 0, 2)


# speedup vs baseline: 1.0120x; 1.0120x over previous
import functools

import jax
import jax.numpy as jnp
from jax import lax
from jax.experimental import pallas as pl
from jax.experimental.pallas import tpu as pltpu
from jax.experimental.pallas import tpu_sc as plsc

NUM_GENRES = 1000000
EMB_DIM = 10

_BL = 131072


def _proj_body(w_ref, b_ref, tab_ref, out_ref):
    t = tab_ref[...]
    w = w_ref[...]
    out_ref[...] = jnp.sum(t * w, axis=0) + b_ref[0]


def _project(table, fc_w, fc_b):
    tab_t = table.T
    grid = (pl.cdiv(NUM_GENRES, _BL),)
    return pl.pallas_call(
        _proj_body,
        grid=grid,
        in_specs=[
            pl.BlockSpec((EMB_DIM, 1), lambda i: (0, 0)),
            pl.BlockSpec(memory_space=pltpu.SMEM),
            pl.BlockSpec((EMB_DIM, _BL), lambda i: (0, i)),
        ],
        out_specs=pl.BlockSpec((_BL,), lambda i: (i,)),
        out_shape=jax.ShapeDtypeStruct((NUM_GENRES,), jnp.float32),
    )(fc_w.reshape(EMB_DIM, 1), fc_b, tab_t)


def _make_gather(n_idx):
    info = plsc.get_sparse_core_info()
    nw = info.num_cores * info.num_subcores
    per_w = n_idx // nw
    chunk = 12800
    n_chunks = per_w // chunk
    mesh = plsc.VectorSubcoreMesh(core_axis_name="c", subcore_axis_name="s")

    @functools.partial(
        pl.kernel,
        mesh=mesh,
        out_type=jax.ShapeDtypeStruct((n_idx,), jnp.float32),
        scratch_types=[
            pltpu.VMEM_SHARED((NUM_GENRES,), jnp.float32),
            pltpu.VMEM((chunk,), jnp.int32),
            pltpu.VMEM((chunk,), jnp.int32),
            pltpu.VMEM((chunk,), jnp.float32),
            pltpu.VMEM((chunk,), jnp.float32),
            pltpu.SemaphoreType.DMA,
            pltpu.SemaphoreType.DMA,
            pltpu.SemaphoreType.DMA,
            pltpu.SemaphoreType.DMA,
            pltpu.SemaphoreType.DMA,
            pltpu.SemaphoreType.DMA,
        ],
    )
    def gather_k(idx_hbm, proj_hbm, out_hbm,
                 shared, idx_v0, idx_v1, val_v0, val_v1,
                 sem_i0, sem_i1, sem_g0, sem_g1, sem_o0, sem_o1):
        sid = lax.axis_index("s")
        wid = sid * info.num_cores + lax.axis_index("c")
        base = wid * per_w
        idx_v = (idx_v0, idx_v1)
        val_v = (val_v0, val_v1)
        sem_i = (sem_i0, sem_i1)
        sem_g = (sem_g0, sem_g1)
        sem_o = (sem_o0, sem_o1)

        idx_h = [None, None]
        out_h = [None, None]
        idx_h[0] = pltpu.async_copy(
            idx_hbm.at[pl.ds(base, chunk)], idx_v[0], sem_i[0])

        @pl.when(sid == 0)
        def _stage():
            pltpu.sync_copy(proj_hbm, shared)

        plsc.subcore_barrier()

        g_h = [None, None]
        for i in range(n_chunks):
            cur = i & 1
            nxt = 1 - cur
            idx_h[cur].wait()
            if out_h[cur] is not None:
                out_h[cur].wait()
            g_h[cur] = pltpu.async_copy(shared.at[idx_v[cur]], val_v[cur],
                                        sem_g[cur])
            if g_h[nxt] is not None:
                g_h[nxt].wait()
                out_h[nxt] = pltpu.async_copy(
                    val_v[nxt], out_hbm.at[pl.ds(base + (i - 1) * chunk, chunk)],
                    sem_o[nxt])
            if i + 1 < n_chunks:
                idx_h[nxt] = pltpu.async_copy(
                    idx_hbm.at[pl.ds(base + (i + 1) * chunk, chunk)],
                    idx_v[nxt], sem_i[nxt])
        last = (n_chunks - 1) & 1
        g_h[last].wait()
        out_h[last] = pltpu.async_copy(
            val_v[last], out_hbm.at[pl.ds(base + (n_chunks - 1) * chunk, chunk)],
            sem_o[last])
        out_h[0].wait()
        out_h[1].wait()

    return gather_k


def kernel(movie_ids, embedding_table, fc_w, fc_b):
    b, l = movie_ids.shape
    n = b * l
    proj = _project(embedding_table, fc_w, fc_b)
    ids_lin = movie_ids.T.reshape(-1)
    out = _make_gather(n)(ids_lin, proj)
    return out.reshape(l, b, 1).transpose(1, 0, 2)
